# Initial kernel scaffold; baseline (speedup 1.0000x reference)
import jax
import jax.numpy as jnp
from jax.experimental import pallas as pl


def kernel(x, y):
    raise NotImplementedError("write your pallas kernel here")


# speedup vs baseline: 3.2775x; 3.2775x over previous
import jax
import jax.numpy as jnp
from jax import lax
from jax.experimental import pallas as pl
from jax.experimental.pallas import tpu as pltpu
from jax.experimental.pallas import tpu_sc as plsc

_ROWS = 2
_COLS = 128
_LANES = 16


def _sc_gather_add(x_hbm, y_hbm, o_hbm, xv, yv, ov, sem):
    c = lax.axis_index("c")
    s = lax.axis_index("s")

    @pl.when(jnp.logical_and(c == 0, s == 0))
    def _():
        cpx = pltpu.async_copy(x_hbm.at[pl.ds(0, _ROWS)], xv, sem)
        cpy = pltpu.async_copy(y_hbm.at[pl.ds(0, _ROWS)], yv, sem)
        cpx.wait()
        cpy.wait()
        for i in range(_ROWS):
            for j in range(_COLS // _LANES):
                sl = pl.ds(j * _LANES, _LANES)
                ov[i, sl] = xv[i, sl] + yv[i, sl]
        pltpu.sync_copy(ov, o_hbm)


def kernel(x, y):
    f = pl.kernel(
        _sc_gather_add,
        out_type=jax.ShapeDtypeStruct((_ROWS, _COLS), jnp.float32),
        mesh=plsc.VectorSubcoreMesh(core_axis_name="c", subcore_axis_name="s"),
        scratch_types=[
            pltpu.VMEM((_ROWS, _COLS), jnp.float32),
            pltpu.VMEM((_ROWS, _COLS), jnp.float32),
            pltpu.VMEM((_ROWS, _COLS), jnp.float32),
            pltpu.SemaphoreType.DMA,
        ],
    )
    return f(x, y)


# speedup vs baseline: 3.5305x; 1.0772x over previous
import jax
import jax.numpy as jnp
from jax import lax
from jax.experimental import pallas as pl
from jax.experimental.pallas import tpu as pltpu
from jax.experimental.pallas import tpu_sc as plsc

_ROWS = 2
_COLS = 128
_LANES = 16


def _sc_gather_add(x_hbm, y_hbm, o_hbm, xv, yv, sem):
    c = lax.axis_index("c")
    s = lax.axis_index("s")

    @pl.when(jnp.logical_and(c == 0, s == 0))
    def _():
        cpx = pltpu.async_copy(x_hbm.at[pl.ds(0, _ROWS)], xv, sem)
        cpy = pltpu.async_copy(y_hbm.at[pl.ds(0, _ROWS)], yv, sem)
        cpx.wait()
        cpy.wait()
        for i in range(_ROWS):
            for j in range(_COLS // _LANES):
                sl = pl.ds(j * _LANES, _LANES)
                xv[i, sl] = xv[i, sl] + yv[i, sl]
        pltpu.sync_copy(xv, o_hbm)


def kernel(x, y):
    f = pl.kernel(
        _sc_gather_add,
        out_type=jax.ShapeDtypeStruct((_ROWS, _COLS), jnp.float32),
        mesh=plsc.VectorSubcoreMesh(
            core_axis_name="c", subcore_axis_name="s", num_cores=1
        ),
        scratch_types=[
            pltpu.VMEM((_ROWS, _COLS), jnp.float32),
            pltpu.VMEM((_ROWS, _COLS), jnp.float32),
            pltpu.SemaphoreType.DMA,
        ],
    )
    return f(x, y)


# speedup vs baseline: 24.7145x; 7.0004x over previous
import jax
import jax.numpy as jnp
from jax.experimental import pallas as pl


def _tc_body(x_ref, y_ref, o_ref):
    o_ref[...] = x_ref[...] + y_ref[...]


def kernel(x, y):
    out = pl.pallas_call(
        _tc_body,
        out_shape=jax.ShapeDtypeStruct((8, 128), jnp.float32),
        grid=(1,),
        in_specs=[
            pl.BlockSpec((8, 128), lambda i: (0, 0)),
            pl.BlockSpec((8, 128), lambda i: (0, 0)),
        ],
        out_specs=pl.BlockSpec((8, 128), lambda i: (0, 0)),
    )(x, y)
    return out[:2]
